# baseline (speedup 1.0000x reference)
import jax
import jax.numpy as jnp
from jax import lax
from jax.experimental import pallas as pl


_BH = 16


def _disp_block(x):
    c = x.shape[-1]
    m = jnp.max(x, axis=-1)
    iota = lax.broadcasted_iota(jnp.int32, x.shape, 2).astype(jnp.float32)
    idx = jnp.min(jnp.where(x == m[..., None], iota, float(c)), axis=-1)
    im1 = (idx - 1.0)[..., None]
    ip1 = (idx + 1.0)[..., None]
    low = jnp.sum(jnp.where(iota == im1, x, 0.0), axis=-1)
    up = jnp.sum(jnp.where(iota == ip1, x, 0.0), axis=-1)
    g = jnp.maximum(low, up)
    nbr = jnp.where(up > low, idx + 1.0, idx - 1.0)
    fl = (m * idx + g * nbr) / (m + g)
    return fl * jnp.float32(0.035) - jnp.float32(4.0)


def _tc_kernel(prob_ref, out_ref):
    out_ref[...] = _disp_block(prob_ref[...])


def kernel(prob):
    hei, wid, cls = prob.shape
    grid = hei // _BH
    return pl.pallas_call(
        _tc_kernel,
        grid=(grid,),
        in_specs=[pl.BlockSpec((_BH, wid, cls), lambda i: (i, 0, 0))],
        out_specs=pl.BlockSpec((_BH, wid), lambda i: (i, 0)),
        out_shape=jax.ShapeDtypeStruct((hei, wid), jnp.float32),
    )(prob)


# speedup vs baseline: 1.5562x; 1.5562x over previous
import jax
import jax.numpy as jnp
from jax import lax
from jax.experimental import pallas as pl


_BH = 16


def _tc_kernel(prob_ref, out_ref):
    x = prob_ref[...]
    xt = jnp.swapaxes(x, 1, 2)
    c = xt.shape[1]
    m = jnp.max(xt, axis=1)
    iota = lax.broadcasted_iota(jnp.int32, xt.shape, 1)
    idx = jnp.min(jnp.where(xt == m[:, None, :], iota, c), axis=1)
    low = jnp.sum(jnp.where(iota == (idx - 1)[:, None, :], xt, 0.0), axis=1)
    up = jnp.sum(jnp.where(iota == (idx + 1)[:, None, :], xt, 0.0), axis=1)
    g = jnp.maximum(low, up)
    idx_f = idx.astype(jnp.float32)
    nbr = jnp.where(up > low, idx_f + 1.0, idx_f - 1.0)
    fl = (m * idx_f + g * nbr) / (m + g)
    out_ref[...] = fl * jnp.float32(0.035) - jnp.float32(4.0)


def kernel(prob):
    hei, wid, cls = prob.shape
    grid = hei // _BH
    return pl.pallas_call(
        _tc_kernel,
        grid=(grid,),
        in_specs=[pl.BlockSpec((_BH, wid, cls), lambda i: (i, 0, 0))],
        out_specs=pl.BlockSpec((_BH, wid), lambda i: (i, 0)),
        out_shape=jax.ShapeDtypeStruct((hei, wid), jnp.float32),
    )(prob)


# speedup vs baseline: 1.5765x; 1.0130x over previous
import jax
import jax.numpy as jnp
from jax import lax
from jax.experimental import pallas as pl


_BH = 32


def _tc_kernel(prob_ref, out_ref):
    x = prob_ref[...]
    xt = jnp.swapaxes(x, 1, 2)
    c = xt.shape[1]
    m = jnp.max(xt, axis=1)
    iota = lax.broadcasted_iota(jnp.int32, xt.shape, 1)
    idx = jnp.min(jnp.where(xt == m[:, None, :], iota, c), axis=1)
    low = jnp.sum(jnp.where(iota == (idx - 1)[:, None, :], xt, 0.0), axis=1)
    up = jnp.sum(jnp.where(iota == (idx + 1)[:, None, :], xt, 0.0), axis=1)
    g = jnp.maximum(low, up)
    idx_f = idx.astype(jnp.float32)
    nbr = jnp.where(up > low, idx_f + 1.0, idx_f - 1.0)
    fl = (m * idx_f + g * nbr) / (m + g)
    out_ref[...] = fl * jnp.float32(0.035) - jnp.float32(4.0)


def kernel(prob):
    hei, wid, cls = prob.shape
    grid = hei // _BH
    return pl.pallas_call(
        _tc_kernel,
        grid=(grid,),
        in_specs=[pl.BlockSpec((_BH, wid, cls), lambda i: (i, 0, 0))],
        out_specs=pl.BlockSpec((_BH, wid), lambda i: (i, 0)),
        out_shape=jax.ShapeDtypeStruct((hei, wid), jnp.float32),
    )(prob)
